# Initial kernel scaffold; baseline (speedup 1.0000x reference)
import jax
import jax.numpy as jnp
from jax.experimental import pallas as pl


def kernel(x, W_g, W_s):
    raise NotImplementedError("write your pallas kernel here")


# speedup vs baseline: 1.8085x; 1.8085x over previous
import functools

import jax
import jax.numpy as jnp
from jax import lax
from jax.experimental import pallas as pl
from jax.experimental.pallas import tpu as pltpu
from jax.experimental.pallas import tpu_sc as plsc

_VOCAB = 100000
_VDIM = 128
_NS = 8
_NCTX = 21
_B = 16384
_NW = 32
_RPW = _B // _NW
_CH = 32
_NCHUNK = _RPW // _CH
_CTXROWS = _CH * _NCTX


_sc_mesh = plsc.VectorSubcoreMesh(core_axis_name="c", subcore_axis_name="s")


@functools.partial(
    pl.kernel,
    out_type=[
        jax.ShapeDtypeStruct((_B, _VDIM), jnp.float32),
        jax.ShapeDtypeStruct((_B, _NS * _VDIM), jnp.float32),
        jax.ShapeDtypeStruct((_B, _VDIM), jnp.float32),
        jax.ShapeDtypeStruct((_NW, _NS * _VDIM), jnp.float32),
    ],
    mesh=_sc_mesh,
    scratch_types=[
        pltpu.VMEM((_CTXROWS,), jnp.int32),
        pltpu.VMEM((_CH,), jnp.int32),
        pltpu.VMEM((_CH,), jnp.int32),
        pltpu.VMEM((_CTXROWS, _VDIM), jnp.float32),
        pltpu.VMEM((_CH, _NS * _VDIM), jnp.float32),
        pltpu.VMEM((_CH, _VDIM), jnp.float32),
        pltpu.VMEM((_CH, _VDIM), jnp.float32),
        pltpu.VMEM((_NS * _VDIM,), jnp.float32),
        pltpu.SemaphoreType.DMA,
    ],
)
def _sc_gather(idx_ctx_hbm, idx_t_hbm, idx_c_hbm, wg_hbm, ws_hbm,
               sc_out, sv_out, cg_out, ssq_out,
               idx_ctx_v, idx_t_v, idx_c_v, ctx_rows, sv_rows, cg_rows,
               acc, ssq, sem):
    wid = lax.axis_index("s") * 2 + lax.axis_index("c")
    base = wid * _RPW

    zero16 = jnp.zeros((16,), jnp.float32)
    for v in range(_NS * _VDIM // 16):
        ssq[pl.ds(v * 16, 16)] = zero16

    def chunk_body(c, carry):
        row0 = base + c * _CH
        pltpu.sync_copy(idx_ctx_hbm.at[pl.ds(row0 * _NCTX, _CTXROWS)],
                        idx_ctx_v)
        pltpu.sync_copy(idx_t_hbm.at[pl.ds(row0, _CH)], idx_t_v)
        pltpu.sync_copy(idx_c_hbm.at[pl.ds(row0, _CH)], idx_c_v)
        cps = []
        for k in range((_CTXROWS + 127) // 128):
            sz = min(128, _CTXROWS - k * 128)
            cps.append(pltpu.async_copy(
                wg_hbm.at[idx_ctx_v.at[pl.ds(k * 128, sz)]],
                ctx_rows.at[pl.ds(k * 128, sz)], sem))
        cps.append(pltpu.async_copy(ws_hbm.at[idx_t_v], sv_rows, sem))
        cps.append(pltpu.async_copy(wg_hbm.at[idx_c_v], cg_rows, sem))
        for cp in cps:
            cp.wait()

        def row_body(r, carry2):
            b0 = r * _NCTX
            for v in range(_VDIM // 16):
                s = ctx_rows[b0, pl.ds(v * 16, 16)]
                for j in range(1, _NCTX):
                    s = s + ctx_rows[b0 + j, pl.ds(v * 16, 16)]
                acc[r, pl.ds(v * 16, 16)] = s
            return carry2

        lax.fori_loop(0, _CH, row_body, 0)

        def ssq_body(r, carry2):
            for v in range(_NS * _VDIM // 16):
                val = sv_rows[r, pl.ds(v * 16, 16)]
                ssq[pl.ds(v * 16, 16)] = ssq[pl.ds(v * 16, 16)] + val * val
            return carry2

        lax.fori_loop(0, _CH, ssq_body, 0)

        pltpu.sync_copy(acc, sc_out.at[pl.ds(row0, _CH)])
        pltpu.sync_copy(sv_rows, sv_out.at[pl.ds(row0, _CH)])
        pltpu.sync_copy(cg_rows, cg_out.at[pl.ds(row0, _CH)])
        return carry

    lax.fori_loop(0, _NCHUNK, chunk_body, 0)
    pltpu.sync_copy(ssq, ssq_out.at[wid])


def _tc_body(ssq_ref, sc_ref, sv_ref, cg_ref, out_ref):
    ssq = jnp.sum(ssq_ref[...], axis=0)
    inv = lax.rsqrt(jnp.maximum(ssq, 1e-12))
    t = sc_ref[...]
    best = jnp.sum(sv_ref[:, 0, :] * (t * inv[0][None, :]),
                   axis=1, keepdims=True)
    besti = jnp.zeros_like(best, dtype=jnp.int32)
    for s in range(1, _NS):
        scr = jnp.sum(sv_ref[:, s, :] * (t * inv[s][None, :]),
                      axis=1, keepdims=True)
        m = scr > best
        best = jnp.where(m, scr, best)
        besti = jnp.where(m, s, besti)
    sel = jnp.zeros_like(t)
    for s in range(_NS):
        sel = sel + jnp.where(besti == s, 1.0, 0.0) * sv_ref[:, s, :]
    dot = jnp.sum(sel * cg_ref[...], axis=1, keepdims=True)
    out_ref[...] = 1.0 / (1.0 + jnp.exp(-dot))


def _tc_finish(ssq, sc, sv, cg):
    bb = 1024
    return pl.pallas_call(
        _tc_body,
        grid=(_B // bb,),
        in_specs=[
            pl.BlockSpec((_NW, _NS, _VDIM), lambda i: (0, 0, 0)),
            pl.BlockSpec((bb, _VDIM), lambda i: (i, 0)),
            pl.BlockSpec((bb, _NS, _VDIM), lambda i: (i, 0, 0)),
            pl.BlockSpec((bb, _VDIM), lambda i: (i, 0)),
        ],
        out_specs=pl.BlockSpec((bb, 1), lambda i: (i, 0)),
        out_shape=jax.ShapeDtypeStruct((_B, 1), jnp.float32),
    )(ssq, sc, sv, cg)


def kernel(x, W_g, W_s):
    idx_t = x[:, 0]
    idx_c = x[:, 1]
    idx_ctx = x[:, 2:].reshape(-1)
    ws2 = W_s.reshape(_VOCAB, _NS * _VDIM)
    sc, sv, cg, ssq = _sc_gather(idx_ctx, idx_t, idx_c, W_g, ws2)
    return _tc_finish(ssq.reshape(_NW, _NS, _VDIM), sc,
                      sv.reshape(_B, _NS, _VDIM), cg)


# speedup vs baseline: 2.5413x; 1.4051x over previous
import functools

import jax
import jax.numpy as jnp
from jax import lax
from jax.experimental import pallas as pl
from jax.experimental.pallas import tpu as pltpu
from jax.experimental.pallas import tpu_sc as plsc

_VOCAB = 100000
_VDIM = 128
_NS = 8
_NCTX = 21
_B = 16384
_NW = 32
_RPW = _B // _NW
_CH = 16
_NCHUNK = _RPW // _CH
_CTXROWS = _CH * _NCTX
_SVROWS = _CH * _NS


_sc_mesh = plsc.VectorSubcoreMesh(core_axis_name="c", subcore_axis_name="s")


@functools.partial(
    pl.kernel,
    out_type=[
        jax.ShapeDtypeStruct((_B, _VDIM), jnp.float32),
        jax.ShapeDtypeStruct((_B * _NS, _VDIM), jnp.float32),
        jax.ShapeDtypeStruct((_B, _VDIM), jnp.float32),
        jax.ShapeDtypeStruct((_NW * _NS, _VDIM), jnp.float32),
    ],
    mesh=_sc_mesh,
    scratch_types=[
        pltpu.VMEM((_CTXROWS,), jnp.int32),
        pltpu.VMEM((_SVROWS,), jnp.int32),
        pltpu.VMEM((_CH,), jnp.int32),
        pltpu.VMEM((_CTXROWS, _VDIM), jnp.float32),
        pltpu.VMEM((_SVROWS, _VDIM), jnp.float32),
        pltpu.VMEM((_CH, _VDIM), jnp.float32),
        pltpu.VMEM((_CH, _VDIM), jnp.float32),
        pltpu.VMEM((_NS, _VDIM), jnp.float32),
        pltpu.SemaphoreType.DMA,
    ],
)
def _sc_gather(idx_ctx_hbm, idx_t8_hbm, idx_c_hbm, wg_hbm, ws_hbm,
               sc_out, sv_out, cg_out, ssq_out,
               idx_ctx_v, idx_sv_v, idx_c_v, ctx_rows, sv_rows,
               cg_rows, acc, ssq, sem):
    wid = lax.axis_index("s") * 2 + lax.axis_index("c")
    base = wid * _RPW

    zero16 = jnp.zeros((16,), jnp.float32)
    for s in range(_NS):
        for v in range(_VDIM // 16):
            ssq[s, pl.ds(v * 16, 16)] = zero16

    def chunk_body(c, carry):
        row0 = base + c * _CH
        pltpu.sync_copy(idx_ctx_hbm.at[pl.ds(row0 * _NCTX, _CTXROWS)],
                        idx_ctx_v)
        pltpu.sync_copy(idx_t8_hbm.at[pl.ds(row0 * _NS, _SVROWS)], idx_sv_v)
        pltpu.sync_copy(idx_c_hbm.at[pl.ds(row0, _CH)], idx_c_v)
        cps = []
        for k in range((_CTXROWS + 127) // 128):
            sz = min(128, _CTXROWS - k * 128)
            cps.append(pltpu.async_copy(
                wg_hbm.at[idx_ctx_v.at[pl.ds(k * 128, sz)]],
                ctx_rows.at[pl.ds(k * 128, sz)], sem))
        cps.append(pltpu.async_copy(ws_hbm.at[idx_sv_v], sv_rows, sem))
        cps.append(pltpu.async_copy(wg_hbm.at[idx_c_v], cg_rows, sem))
        for cp in cps:
            cp.wait()

        def row_body(r, carry2):
            b0 = r * _NCTX
            for v in range(_VDIM // 16):
                s = ctx_rows[b0, pl.ds(v * 16, 16)]
                for j in range(1, _NCTX):
                    s = s + ctx_rows[b0 + j, pl.ds(v * 16, 16)]
                acc[r, pl.ds(v * 16, 16)] = s
            return carry2

        lax.fori_loop(0, _CH, row_body, 0)

        def ssq_body(r, carry2):
            for s in range(_NS):
                for v in range(_VDIM // 16):
                    val = sv_rows[r * _NS + s, pl.ds(v * 16, 16)]
                    ssq[s, pl.ds(v * 16, 16)] = (
                        ssq[s, pl.ds(v * 16, 16)] + val * val)
            return carry2

        lax.fori_loop(0, _CH, ssq_body, 0)

        pltpu.sync_copy(acc, sc_out.at[pl.ds(row0, _CH)])
        pltpu.sync_copy(sv_rows, sv_out.at[pl.ds(row0 * _NS, _SVROWS)])
        pltpu.sync_copy(cg_rows, cg_out.at[pl.ds(row0, _CH)])
        return carry

    lax.fori_loop(0, _NCHUNK, chunk_body, 0)
    pltpu.sync_copy(ssq, ssq_out.at[pl.ds(wid * _NS, _NS)])


def _tc_body(ssq_ref, sc_ref, sv_ref, cg_ref, out_ref):
    ssq = jnp.sum(ssq_ref[...], axis=0)
    inv = lax.rsqrt(jnp.maximum(ssq, 1e-12))
    t = sc_ref[...]
    best = jnp.sum(sv_ref[:, 0, :] * (t * inv[0][None, :]),
                   axis=1, keepdims=True)
    besti = jnp.zeros_like(best, dtype=jnp.int32)
    for s in range(1, _NS):
        scr = jnp.sum(sv_ref[:, s, :] * (t * inv[s][None, :]),
                      axis=1, keepdims=True)
        m = scr > best
        best = jnp.where(m, scr, best)
        besti = jnp.where(m, s, besti)
    sel = jnp.zeros_like(t)
    for s in range(_NS):
        sel = sel + jnp.where(besti == s, 1.0, 0.0) * sv_ref[:, s, :]
    dot = jnp.sum(sel * cg_ref[...], axis=1, keepdims=True)
    out_ref[...] = 1.0 / (1.0 + jnp.exp(-dot))


def _tc_finish(ssq, sc, sv, cg):
    bb = 1024
    return pl.pallas_call(
        _tc_body,
        grid=(_B // bb,),
        in_specs=[
            pl.BlockSpec((_NW, _NS, _VDIM), lambda i: (0, 0, 0)),
            pl.BlockSpec((bb, _VDIM), lambda i: (i, 0)),
            pl.BlockSpec((bb, _NS, _VDIM), lambda i: (i, 0, 0)),
            pl.BlockSpec((bb, _VDIM), lambda i: (i, 0)),
        ],
        out_specs=pl.BlockSpec((bb, 1), lambda i: (i, 0)),
        out_shape=jax.ShapeDtypeStruct((_B, 1), jnp.float32),
    )(ssq, sc, sv, cg)


def kernel(x, W_g, W_s):
    idx_ctx = x[:, 2:].reshape(-1)
    idx_t8 = (x[:, 0:1] * _NS + jnp.arange(_NS, dtype=x.dtype)).reshape(-1)
    idx_c = x[:, 1]
    sc, sv, cg, ssq = _sc_gather(idx_ctx, idx_t8, idx_c, W_g,
                                 W_s.reshape(_VOCAB * _NS, _VDIM))
    return _tc_finish(ssq.reshape(_NW, _NS, _VDIM), sc,
                      sv.reshape(_B, _NS, _VDIM), cg)


# speedup vs baseline: 2.8934x; 1.1386x over previous
import functools

import jax
import jax.numpy as jnp
from jax import lax
from jax.experimental import pallas as pl
from jax.experimental.pallas import tpu as pltpu
from jax.experimental.pallas import tpu_sc as plsc

_VOCAB = 100000
_VDIM = 128
_NS = 8
_NCTX = 21
_B = 16384
_NW = 32
_RPW = _B // _NW
_CH = 16
_NCHUNK = _RPW // _CH
_CTXROWS = _CH * _NCTX
_SVROWS = _CH * _NS


_sc_mesh = plsc.VectorSubcoreMesh(core_axis_name="c", subcore_axis_name="s")


@functools.partial(
    pl.kernel,
    out_type=[
        jax.ShapeDtypeStruct((_B, _VDIM), jnp.float32),
        jax.ShapeDtypeStruct((_B * _NS, _VDIM), jnp.float32),
        jax.ShapeDtypeStruct((_B, _VDIM), jnp.float32),
        jax.ShapeDtypeStruct((_NW * _NS, _VDIM), jnp.float32),
    ],
    mesh=_sc_mesh,
    scratch_types=[
        pltpu.VMEM((_CTXROWS,), jnp.int32),
        pltpu.VMEM((_CTXROWS,), jnp.int32),
        pltpu.VMEM((_SVROWS,), jnp.int32),
        pltpu.VMEM((_SVROWS,), jnp.int32),
        pltpu.VMEM((_CH,), jnp.int32),
        pltpu.VMEM((_CH,), jnp.int32),
        pltpu.VMEM((2, _CTXROWS, _VDIM), jnp.float32),
        pltpu.VMEM((2, _SVROWS, _VDIM), jnp.float32),
        pltpu.VMEM((2, _CH, _VDIM), jnp.float32),
        pltpu.VMEM((2, _CH, _VDIM), jnp.float32),
        pltpu.VMEM((_NS, _VDIM), jnp.float32),
        pltpu.SemaphoreType.DMA,
        pltpu.SemaphoreType.DMA,
        pltpu.SemaphoreType.DMA,
        pltpu.SemaphoreType.DMA,
        pltpu.SemaphoreType.DMA,
        pltpu.SemaphoreType.DMA,
    ],
)
def _sc_gather(idx_ctx_hbm, idx_t8_hbm, idx_c_hbm, wg_hbm, ws_hbm,
               sc_out, sv_out, cg_out, ssq_out,
               idx_ctx_a, idx_ctx_b, idx_sv_a, idx_sv_b, idx_c_a, idx_c_b,
               ctx_rows, sv_rows, cg_rows, acc, ssq,
               sem_ga, sem_gb, sem_ia, sem_ib, sem_wa, sem_wb):
    wid = lax.axis_index("s") * 2 + lax.axis_index("c")
    base = wid * _RPW
    nsplit = (_CTXROWS + 127) // 128
    idx_ctx_p = (idx_ctx_a, idx_ctx_b)
    idx_sv_p = (idx_sv_a, idx_sv_b)
    idx_c_p = (idx_c_a, idx_c_b)

    zero16 = jnp.zeros((16,), jnp.float32)
    for s in range(_NS):
        for v in range(_VDIM // 16):
            ssq[s, pl.ds(v * 16, 16)] = zero16

    def idx_copies(chunk, p, sem):
        row0 = base + chunk * _CH
        return [
            pltpu.make_async_copy(
                idx_ctx_hbm.at[pl.ds(row0 * _NCTX, _CTXROWS)],
                idx_ctx_p[p], sem),
            pltpu.make_async_copy(
                idx_t8_hbm.at[pl.ds(row0 * _NS, _SVROWS)],
                idx_sv_p[p], sem),
            pltpu.make_async_copy(
                idx_c_hbm.at[pl.ds(row0, _CH)], idx_c_p[p], sem),
        ]

    def gather_copies(p, sem):
        cps = []
        for k in range(nsplit):
            sz = min(128, _CTXROWS - k * 128)
            cps.append(pltpu.make_async_copy(
                wg_hbm.at[idx_ctx_p[p].at[pl.ds(k * 128, sz)]],
                ctx_rows.at[p, pl.ds(k * 128, sz)], sem))
        cps.append(pltpu.make_async_copy(
            ws_hbm.at[idx_sv_p[p]], sv_rows.at[p], sem))
        cps.append(pltpu.make_async_copy(
            wg_hbm.at[idx_c_p[p]], cg_rows.at[p], sem))
        return cps

    def wb_copies(chunk, p, sem):
        row0 = base + chunk * _CH
        return [
            pltpu.make_async_copy(acc.at[p], sc_out.at[pl.ds(row0, _CH)],
                                  sem),
            pltpu.make_async_copy(
                sv_rows.at[p], sv_out.at[pl.ds(row0 * _NS, _SVROWS)], sem),
            pltpu.make_async_copy(cg_rows.at[p],
                                  cg_out.at[pl.ds(row0, _CH)], sem),
        ]

    def start(cps):
        for cp in cps:
            cp.start()

    def wait(cps):
        for cp in cps:
            cp.wait()

    def compute(p):
        def row_body(r, carry2):
            b0 = r * _NCTX
            for v in range(_VDIM // 16):
                s = ctx_rows[p, b0, pl.ds(v * 16, 16)]
                for j in range(1, _NCTX):
                    s = s + ctx_rows[p, b0 + j, pl.ds(v * 16, 16)]
                acc[p, r, pl.ds(v * 16, 16)] = s
            return carry2

        lax.fori_loop(0, _CH, row_body, 0)

        def ssq_body(r, carry2):
            for s in range(_NS):
                for v in range(_VDIM // 16):
                    val = sv_rows[p, r * _NS + s, pl.ds(v * 16, 16)]
                    ssq[s, pl.ds(v * 16, 16)] = (
                        ssq[s, pl.ds(v * 16, 16)] + val * val)
            return carry2

        lax.fori_loop(0, _CH, ssq_body, 0)

    start(idx_copies(0, 0, sem_ia))
    wait(idx_copies(0, 0, sem_ia))
    start(gather_copies(0, sem_ga))
    start(idx_copies(1, 1, sem_ib))

    def pipe_body(i, carry):
        ca = 2 * i

        @pl.when(i > 0)
        def _():
            wait(wb_copies(ca - 1, 1, sem_wb))

        wait(idx_copies(ca + 1, 1, sem_ib))
        start(gather_copies(1, sem_gb))
        wait(gather_copies(0, sem_ga))

        @pl.when(ca + 2 < _NCHUNK)
        def _():
            start(idx_copies(ca + 2, 0, sem_ia))

        compute(0)
        start(wb_copies(ca, 0, sem_wa))
        wait(gather_copies(1, sem_gb))
        compute(1)
        start(wb_copies(ca + 1, 1, sem_wb))
        wait(wb_copies(ca, 0, sem_wa))

        @pl.when(ca + 2 < _NCHUNK)
        def _():
            wait(idx_copies(ca + 2, 0, sem_ia))
            start(gather_copies(0, sem_ga))

        @pl.when(ca + 3 < _NCHUNK)
        def _():
            start(idx_copies(ca + 3, 1, sem_ib))

        return carry

    lax.fori_loop(0, _NCHUNK // 2, pipe_body, 0)
    wait(wb_copies(_NCHUNK - 1, 1, sem_wb))
    pltpu.sync_copy(ssq, ssq_out.at[pl.ds(wid * _NS, _NS)])


def _tc_body(ssq_ref, sc_ref, sv_ref, cg_ref, out_ref):
    ssq = jnp.sum(ssq_ref[...], axis=0)
    inv = lax.rsqrt(jnp.maximum(ssq, 1e-12))
    t = sc_ref[...]
    best = jnp.sum(sv_ref[:, 0, :] * (t * inv[0][None, :]),
                   axis=1, keepdims=True)
    besti = jnp.zeros_like(best, dtype=jnp.int32)
    for s in range(1, _NS):
        scr = jnp.sum(sv_ref[:, s, :] * (t * inv[s][None, :]),
                      axis=1, keepdims=True)
        m = scr > best
        best = jnp.where(m, scr, best)
        besti = jnp.where(m, s, besti)
    sel = jnp.zeros_like(t)
    for s in range(_NS):
        sel = sel + jnp.where(besti == s, 1.0, 0.0) * sv_ref[:, s, :]
    dot = jnp.sum(sel * cg_ref[...], axis=1, keepdims=True)
    out_ref[...] = 1.0 / (1.0 + jnp.exp(-dot))


def _tc_finish(ssq, sc, sv, cg):
    bb = 1024
    return pl.pallas_call(
        _tc_body,
        grid=(_B // bb,),
        in_specs=[
            pl.BlockSpec((_NW, _NS, _VDIM), lambda i: (0, 0, 0)),
            pl.BlockSpec((bb, _VDIM), lambda i: (i, 0)),
            pl.BlockSpec((bb, _NS, _VDIM), lambda i: (i, 0, 0)),
            pl.BlockSpec((bb, _VDIM), lambda i: (i, 0)),
        ],
        out_specs=pl.BlockSpec((bb, 1), lambda i: (i, 0)),
        out_shape=jax.ShapeDtypeStruct((_B, 1), jnp.float32),
    )(ssq, sc, sv, cg)


def kernel(x, W_g, W_s):
    idx_ctx = x[:, 2:].reshape(-1)
    idx_t8 = (x[:, 0:1] * _NS + jnp.arange(_NS, dtype=x.dtype)).reshape(-1)
    idx_c = x[:, 1]
    sc, sv, cg, ssq = _sc_gather(idx_ctx, idx_t8, idx_c, W_g,
                                 W_s.reshape(_VOCAB * _NS, _VDIM))
    return _tc_finish(ssq.reshape(_NW, _NS, _VDIM), sc,
                      sv.reshape(_B, _NS, _VDIM), cg)
